# baseline (speedup 1.0000x reference)
import functools

import jax
import jax.numpy as jnp
import numpy as np
from jax import lax
from jax.experimental import pallas as pl
from jax.experimental.pallas import tpu as pltpu
from jax.experimental.pallas import tpu_sc as plsc

_CELL = 0.2
_MIN_R = np.array([-40.0, -40.0, -1.0], dtype=np.float32)
_TMAX = np.array([399, 399, 21], dtype=np.int32)

_N = 20000
_BQ = 256
_BK = 2048
_NG = _BK // 128
_NQ = 20480
_NK = 20480
_NQB = _NQ // _BQ
_NKB = _NK // _BK

_PAD_KN = 8.0e6
_D_OFF = 1 << 20
_INIT_P = 0x7FFFFFFF

_PER_TILE = _NQ // 32
_LANES = 16


def _nn_a_body(qa_ref, ka_ref, jout_ref, rp, rk):
    k = pl.program_id(1)

    @pl.when(k == 0)
    def _init():
        rp[...] = jnp.full((_BQ, 128), _INIT_P, jnp.int32)
        rk[...] = jnp.zeros((_BQ, 128), jnp.int32)

    d = jnp.dot(qa_ref[...], ka_ref[...],
                preferred_element_type=jnp.float32)
    m1 = (d[:, 0:128].astype(jnp.int32) * 128
          + jnp.int32(_D_OFF * 128))
    for g in range(1, _NG):
        m1 = jnp.minimum(
            m1, d[:, g * 128:(g + 1) * 128].astype(jnp.int32) * 128
            + jnp.int32(_D_OFF * 128 + g))
    dm1 = m1 >> 7
    g1 = m1 & 127
    better = dm1 < rp[...]
    rp[...] = jnp.minimum(dm1, rp[...])
    rk[...] = jnp.where(better, jnp.int32(k * _NG) + g1, rk[...])

    @pl.when(k == _NKB - 1)
    def _finalize():
        rdv = rp[...]
        lane = lax.broadcasted_iota(jnp.int32, (_BQ, 128), 1)
        rj = rk[...] * 128 + lane
        dmin = jnp.min(rdv, axis=1, keepdims=True)
        att = rdv == dmin
        jout_ref[...] = jnp.min(
            jnp.where(att, rj, jnp.int32(_INIT_P)),
            axis=1, keepdims=True).astype(jnp.float32)


def _nn_b_body(qa_ref, ka_ref, nqn_ref, mout_ref, rg, rk):
    k = pl.program_id(1)

    @pl.when(k == 0)
    def _init():
        rg[...] = jnp.full((_BQ, 128), -1, jnp.int32)
        rk[...] = jnp.zeros((_BQ, 128), jnp.int32)

    d = jnp.dot(qa_ref[...], ka_ref[...],
                preferred_element_type=jnp.float32)
    nqn = nqn_ref[...]
    m1 = jnp.where(d[:, 0:128] == nqn, jnp.int32(0), jnp.int32(-1))
    for g in range(1, _NG):
        m1 = jnp.maximum(
            m1, jnp.where(d[:, g * 128:(g + 1) * 128] == nqn,
                          jnp.int32(g), jnp.int32(-1)))
    upd = m1 >= 0
    rg[...] = jnp.where(upd, m1, rg[...])
    rk[...] = jnp.where(upd, jnp.int32(k), rk[...])

    @pl.when(k == _NKB - 1)
    def _finalize():
        lane = lax.broadcasted_iota(jnp.int32, (_BQ, 128), 1)
        rgv = rg[...]
        rj = rk[...] * _BK + rgv * 128 + lane
        rj = jnp.where(rgv >= 0, rj, jnp.int32(-1))
        mout_ref[...] = jnp.max(rj, axis=1, keepdims=True).astype(jnp.float32)


_nn_a = pl.pallas_call(
    _nn_a_body,
    grid=(_NQB, _NKB),
    in_specs=[
        pl.BlockSpec((_BQ, 8), lambda i, k: (i, 0)),
        pl.BlockSpec((8, _BK), lambda i, k: (0, k)),
    ],
    out_specs=pl.BlockSpec((_BQ, 1), lambda i, k: (i, 0)),
    out_shape=jax.ShapeDtypeStruct((_NQ, 1), jnp.float32),
    scratch_shapes=[pltpu.VMEM((_BQ, 128), jnp.int32),
                    pltpu.VMEM((_BQ, 128), jnp.int32)],
    compiler_params=pltpu.CompilerParams(
        dimension_semantics=("arbitrary", "arbitrary")),
)

_nn_b = pl.pallas_call(
    _nn_b_body,
    grid=(_NQB, _NKB),
    in_specs=[
        pl.BlockSpec((_BQ, 8), lambda i, k: (i, 0)),
        pl.BlockSpec((8, _BK), lambda i, k: (0, k)),
        pl.BlockSpec((_BQ, 1), lambda i, k: (i, 0)),
    ],
    out_specs=pl.BlockSpec((_BQ, 1), lambda i, k: (i, 0)),
    out_shape=jax.ShapeDtypeStruct((_NQ, 1), jnp.float32),
    scratch_shapes=[pltpu.VMEM((_BQ, 128), jnp.int32),
                    pltpu.VMEM((_BQ, 128), jnp.int32)],
    compiler_params=pltpu.CompilerParams(
        dimension_semantics=("arbitrary", "arbitrary")),
)


def _sc_gather_body(jall, mall, p1x, p1y, p1z, flx, fly, flz,
                    p2x, p2y, p2z, d2_out, nn_out,
                    jst_v, p1x_v, p1y_v, p1z_v, flx_v, fly_v, flz_v,
                    idx_v, rep_v, px_v, py_v, pz_v,
                    d2_v, nn_v, sem):
    wid = lax.axis_index("s") * 2 + lax.axis_index("c")
    base = wid * _PER_TILE
    nsteps = _PER_TILE // _LANES
    pltpu.sync_copy(jall.at[pl.ds(base, _PER_TILE)], jst_v)
    pltpu.sync_copy(p1x.at[pl.ds(base, _PER_TILE)], p1x_v)
    pltpu.sync_copy(p1y.at[pl.ds(base, _PER_TILE)], p1y_v)
    pltpu.sync_copy(p1z.at[pl.ds(base, _PER_TILE)], p1z_v)
    pltpu.sync_copy(flx.at[pl.ds(base, _PER_TILE)], flx_v)
    pltpu.sync_copy(fly.at[pl.ds(base, _PER_TILE)], fly_v)
    pltpu.sync_copy(flz.at[pl.ds(base, _PER_TILE)], flz_v)
    for t in range(nsteps):
        sl = pl.ds(t * _LANES, _LANES)
        idx_v[sl] = jst_v[sl].astype(jnp.int32)
    pltpu.async_copy(mall.at[idx_v], rep_v, sem).wait()
    for t in range(nsteps):
        sl = pl.ds(t * _LANES, _LANES)
        nn_v[sl] = rep_v[sl].astype(jnp.int32)
    cx = pltpu.async_copy(p2x.at[nn_v], px_v, sem)
    cy = pltpu.async_copy(p2y.at[nn_v], py_v, sem)
    cz = pltpu.async_copy(p2z.at[nn_v], pz_v, sem)
    cx.wait()
    cy.wait()
    cz.wait()
    for t in range(nsteps):
        sl = pl.ds(t * _LANES, _LANES)
        dx = px_v[sl] - (p1x_v[sl] + flx_v[sl])
        dy = py_v[sl] - (p1y_v[sl] + fly_v[sl])
        dz = pz_v[sl] - (p1z_v[sl] + flz_v[sl])
        d2_v[sl] = dx * dx + dy * dy + dz * dz
    pltpu.sync_copy(d2_v, d2_out.at[pl.ds(base, _PER_TILE)])
    pltpu.sync_copy(nn_v, nn_out.at[pl.ds(base, _PER_TILE)])


@functools.cache
def _make_sc_gather():
  return pl.kernel(
    _sc_gather_body,
    mesh=plsc.VectorSubcoreMesh(core_axis_name="c", subcore_axis_name="s"),
    out_type=[
        jax.ShapeDtypeStruct((_NQ,), jnp.float32),
        jax.ShapeDtypeStruct((_NQ,), jnp.int32),
    ],
    scratch_types=[
        pltpu.VMEM((_PER_TILE,), jnp.float32),
        pltpu.VMEM((_PER_TILE,), jnp.float32),
        pltpu.VMEM((_PER_TILE,), jnp.float32),
        pltpu.VMEM((_PER_TILE,), jnp.float32),
        pltpu.VMEM((_PER_TILE,), jnp.float32),
        pltpu.VMEM((_PER_TILE,), jnp.float32),
        pltpu.VMEM((_PER_TILE,), jnp.float32),
        pltpu.VMEM((_PER_TILE,), jnp.int32),
        pltpu.VMEM((_PER_TILE,), jnp.float32),
        pltpu.VMEM((_PER_TILE,), jnp.float32),
        pltpu.VMEM((_PER_TILE,), jnp.float32),
        pltpu.VMEM((_PER_TILE,), jnp.float32),
        pltpu.VMEM((_PER_TILE,), jnp.float32),
        pltpu.VMEM((_PER_TILE,), jnp.int32),
        pltpu.SemaphoreType.DMA,
    ],
  )


def _pad_cols(rows):
    out = jnp.zeros((8, _NK), jnp.float32)
    for r, v in enumerate(rows):
        out = out.at[r, :_N].set(v)
    return out


def kernel(pc1, pred_flow, pc2):
    minr = jnp.asarray(_MIN_R)
    tmax = jnp.asarray(_TMAX)
    p2 = pc2[0]
    p1 = pc1[0]
    fl = pred_flow[0]
    kci = jnp.clip(jnp.floor((p2 - minr) / _CELL).astype(jnp.int32), 0, tmax)
    deformed = p1 + fl
    qci = jnp.clip(jnp.floor((deformed - minr) / _CELL).astype(jnp.int32),
                   0, tmax)
    kc = kci.astype(jnp.float32)
    qc = qci.astype(jnp.float32)

    qr = qc.astype(jnp.bfloat16).astype(jnp.float32)
    kr = kc.astype(jnp.bfloat16).astype(jnp.float32)
    kni = jnp.sum(kci * kci, axis=1)
    knH = ((kni >> 11) << 11).astype(jnp.float32)
    knM = (((kni >> 3) & 0xFF) << 3).astype(jnp.float32)
    knL = (kni & 0x7).astype(jnp.float32)
    qaug_a = jnp.zeros((_NQ, 8), jnp.float32)
    qaug_a = qaug_a.at[:_N, 0:3].set(-2.0 * qr)
    qaug_a = qaug_a.at[:_N, 3:6].set(1.0)
    kaug_a = _pad_cols([kr[:, 0], kr[:, 1], kr[:, 2], knH, knM, knL])
    kaug_a = kaug_a.at[3, _N:].set(_PAD_KN)

    kh = (kci >> 4).astype(jnp.float32)
    kl = (kci & 0xF).astype(jnp.float32)
    kn6i = jnp.sum((kci >> 4) ** 2 + (kci & 0xF) ** 2, axis=1)
    kn6H = ((kn6i >> 4) << 4).astype(jnp.float32)
    kn6L = (kn6i & 0xF).astype(jnp.float32)
    qaug_b = jnp.zeros((_NQ, 8), jnp.float32)
    qaug_b = qaug_b.at[:_N, 0].set(-2.0 * kh[:, 0])
    qaug_b = qaug_b.at[:_N, 1].set(-2.0 * kl[:, 0])
    qaug_b = qaug_b.at[:_N, 2].set(-2.0 * kh[:, 1])
    qaug_b = qaug_b.at[:_N, 3].set(-2.0 * kl[:, 1])
    qaug_b = qaug_b.at[:_N, 4].set(-2.0 * kh[:, 2])
    qaug_b = qaug_b.at[:_N, 5].set(-2.0 * kl[:, 2])
    qaug_b = qaug_b.at[:_N, 6:8].set(1.0)
    kaug_b = _pad_cols([kh[:, 0], kl[:, 0], kh[:, 1], kl[:, 1],
                        kh[:, 2], kl[:, 2], kn6H, kn6L])
    kaug_b = kaug_b.at[6, _N:].set(_PAD_KN)
    nqn6 = jnp.zeros((_NQ, 1), jnp.float32)
    nqn6 = nqn6.at[:_N, 0].set(-kn6i.astype(jnp.float32))

    jstar = _nn_a(qaug_a, kaug_a).reshape(_NQ)
    rep = _nn_b(qaug_b, kaug_b, nqn6).reshape(_NQ)

    d2, nn = _make_sc_gather()(
        jstar, rep,
        p1[:, 0], p1[:, 1], p1[:, 2],
        fl[:, 0], fl[:, 1], fl[:, 2],
        p2[:, 0], p2[:, 1], p2[:, 2])

    dist = jnp.sqrt(d2[:_N]).reshape(1, _N)
    return dist, nn[:_N]


# speedup vs baseline: 1.0407x; 1.0407x over previous
import functools

import jax
import jax.numpy as jnp
import numpy as np
from jax import lax
from jax.experimental import pallas as pl
from jax.experimental.pallas import tpu as pltpu
from jax.experimental.pallas import tpu_sc as plsc

_CELL = 0.2
_MIN_R = np.array([-40.0, -40.0, -1.0], dtype=np.float32)
_TMAX = np.array([399, 399, 21], dtype=np.int32)

_N = 20000
_BQ = 256
_BK = 2048
_NG = _BK // 128
_NQ = 20480
_NK = 20480
_NQB = _NQ // _BQ
_NKB = _NK // _BK

_PAD_KN = 8.0e6
_D_OFF = 1 << 20
_INIT_P = 0x7FFFFFFF

_PER_TILE = _NQ // 32
_LANES = 16


def _nn_a_body(qa_ref, ka_ref, jout_ref, rp, rk):
    k = pl.program_id(1)

    @pl.when(k == 0)
    def _init():
        rp[...] = jnp.full((_BQ, 128), _INIT_P, jnp.int32)
        rk[...] = jnp.zeros((_BQ, 128), jnp.int32)

    d = jnp.dot(qa_ref[...], ka_ref[...],
                preferred_element_type=jnp.float32)
    m1 = (d[:, 0:128].astype(jnp.int32) * 128
          + jnp.int32(_D_OFF * 128))
    for g in range(1, _NG):
        m1 = jnp.minimum(
            m1, d[:, g * 128:(g + 1) * 128].astype(jnp.int32) * 128
            + jnp.int32(_D_OFF * 128 + g))
    dm1 = m1 >> 7
    g1 = m1 & 127
    better = dm1 < rp[...]
    rp[...] = jnp.minimum(dm1, rp[...])
    rk[...] = jnp.where(better, jnp.int32(k * _NG) + g1, rk[...])

    @pl.when(k == _NKB - 1)
    def _finalize():
        rdv = rp[...]
        lane = lax.broadcasted_iota(jnp.int32, (_BQ, 128), 1)
        rj = rk[...] * 128 + lane
        dmin = jnp.min(rdv, axis=1, keepdims=True)
        att = rdv == dmin
        jout_ref[...] = jnp.min(
            jnp.where(att, rj, jnp.int32(_INIT_P)),
            axis=1, keepdims=True).astype(jnp.float32)


def _nn_b_body(qa_ref, ka_ref, nqn_ref, mout_ref, rg, rk):
    k = pl.program_id(1)

    @pl.when(k == 0)
    def _init():
        rg[...] = jnp.full((_BQ, 128), -1, jnp.int32)
        rk[...] = jnp.zeros((_BQ, 128), jnp.int32)

    d = jnp.dot(qa_ref[...], ka_ref[...],
                preferred_element_type=jnp.float32)
    nqn = nqn_ref[...]
    m1 = jnp.where(d[:, 0:128] == nqn, jnp.int32(0), jnp.int32(-1))
    for g in range(1, _NG):
        m1 = jnp.maximum(
            m1, jnp.where(d[:, g * 128:(g + 1) * 128] == nqn,
                          jnp.int32(g), jnp.int32(-1)))
    upd = m1 >= 0
    rg[...] = jnp.where(upd, m1, rg[...])
    rk[...] = jnp.where(upd, jnp.int32(k), rk[...])

    @pl.when(k == _NKB - 1)
    def _finalize():
        lane = lax.broadcasted_iota(jnp.int32, (_BQ, 128), 1)
        rgv = rg[...]
        rj = rk[...] * _BK + rgv * 128 + lane
        rj = jnp.where(rgv >= 0, rj, jnp.int32(-1))
        mout_ref[...] = jnp.max(rj, axis=1, keepdims=True).astype(jnp.float32)


_nn_a = pl.pallas_call(
    _nn_a_body,
    grid=(_NQB, _NKB),
    in_specs=[
        pl.BlockSpec((_BQ, 8), lambda i, k: (i, 0)),
        pl.BlockSpec((8, _BK), lambda i, k: (0, k)),
    ],
    out_specs=pl.BlockSpec((_BQ, 1), lambda i, k: (i, 0)),
    out_shape=jax.ShapeDtypeStruct((_NQ, 1), jnp.float32),
    scratch_shapes=[pltpu.VMEM((_BQ, 128), jnp.int32),
                    pltpu.VMEM((_BQ, 128), jnp.int32)],
    compiler_params=pltpu.CompilerParams(
        dimension_semantics=("arbitrary", "arbitrary")),
)

_nn_b = pl.pallas_call(
    _nn_b_body,
    grid=(_NQB, _NKB),
    in_specs=[
        pl.BlockSpec((_BQ, 8), lambda i, k: (i, 0)),
        pl.BlockSpec((8, _BK), lambda i, k: (0, k)),
        pl.BlockSpec((_BQ, 1), lambda i, k: (i, 0)),
    ],
    out_specs=pl.BlockSpec((_BQ, 1), lambda i, k: (i, 0)),
    out_shape=jax.ShapeDtypeStruct((_NQ, 1), jnp.float32),
    scratch_shapes=[pltpu.VMEM((_BQ, 128), jnp.int32),
                    pltpu.VMEM((_BQ, 128), jnp.int32)],
    compiler_params=pltpu.CompilerParams(
        dimension_semantics=("arbitrary", "arbitrary")),
)


def _sc_gather_body(jall, mall, p1x, p1y, p1z, flx, fly, flz,
                    p2x, p2y, p2z, d2_out, nn_out,
                    jst_v, p1x_v, p1y_v, p1z_v, flx_v, fly_v, flz_v,
                    idx_v, rep_v, px_v, py_v, pz_v,
                    d2_v, nn_v, sem):
    wid = lax.axis_index("s") * 2 + lax.axis_index("c")
    base = wid * _PER_TILE
    nsteps = _PER_TILE // _LANES
    pltpu.sync_copy(jall.at[pl.ds(base, _PER_TILE)], jst_v)
    pltpu.sync_copy(p1x.at[pl.ds(base, _PER_TILE)], p1x_v)
    pltpu.sync_copy(p1y.at[pl.ds(base, _PER_TILE)], p1y_v)
    pltpu.sync_copy(p1z.at[pl.ds(base, _PER_TILE)], p1z_v)
    pltpu.sync_copy(flx.at[pl.ds(base, _PER_TILE)], flx_v)
    pltpu.sync_copy(fly.at[pl.ds(base, _PER_TILE)], fly_v)
    pltpu.sync_copy(flz.at[pl.ds(base, _PER_TILE)], flz_v)
    for t in range(nsteps):
        sl = pl.ds(t * _LANES, _LANES)
        idx_v[sl] = jst_v[sl].astype(jnp.int32)
    pltpu.async_copy(mall.at[idx_v], rep_v, sem).wait()
    for t in range(nsteps):
        sl = pl.ds(t * _LANES, _LANES)
        nn_v[sl] = rep_v[sl].astype(jnp.int32)
    cx = pltpu.async_copy(p2x.at[nn_v], px_v, sem)
    cy = pltpu.async_copy(p2y.at[nn_v], py_v, sem)
    cz = pltpu.async_copy(p2z.at[nn_v], pz_v, sem)
    cx.wait()
    cy.wait()
    cz.wait()
    for t in range(nsteps):
        sl = pl.ds(t * _LANES, _LANES)
        dx = px_v[sl] - (p1x_v[sl] + flx_v[sl])
        dy = py_v[sl] - (p1y_v[sl] + fly_v[sl])
        dz = pz_v[sl] - (p1z_v[sl] + flz_v[sl])
        d2_v[sl] = dx * dx + dy * dy + dz * dz
    pltpu.sync_copy(d2_v, d2_out.at[pl.ds(base, _PER_TILE)])
    pltpu.sync_copy(nn_v, nn_out.at[pl.ds(base, _PER_TILE)])


@functools.cache
def _make_sc_gather():
  return pl.kernel(
    _sc_gather_body,
    mesh=plsc.VectorSubcoreMesh(core_axis_name="c", subcore_axis_name="s"),
    out_type=[
        jax.ShapeDtypeStruct((_NQ,), jnp.float32),
        jax.ShapeDtypeStruct((_NQ,), jnp.int32),
    ],
    scratch_types=[
        pltpu.VMEM((_PER_TILE,), jnp.float32),
        pltpu.VMEM((_PER_TILE,), jnp.float32),
        pltpu.VMEM((_PER_TILE,), jnp.float32),
        pltpu.VMEM((_PER_TILE,), jnp.float32),
        pltpu.VMEM((_PER_TILE,), jnp.float32),
        pltpu.VMEM((_PER_TILE,), jnp.float32),
        pltpu.VMEM((_PER_TILE,), jnp.float32),
        pltpu.VMEM((_PER_TILE,), jnp.int32),
        pltpu.VMEM((_PER_TILE,), jnp.float32),
        pltpu.VMEM((_PER_TILE,), jnp.float32),
        pltpu.VMEM((_PER_TILE,), jnp.float32),
        pltpu.VMEM((_PER_TILE,), jnp.float32),
        pltpu.VMEM((_PER_TILE,), jnp.float32),
        pltpu.VMEM((_PER_TILE,), jnp.int32),
        pltpu.SemaphoreType.DMA,
    ],
  )


def _pad_cols(rows):
    out = jnp.zeros((8, _NK), jnp.float32)
    for r, v in enumerate(rows):
        out = out.at[r, :_N].set(v)
    return out


def kernel(pc1, pred_flow, pc2):
    minr = jnp.asarray(_MIN_R)
    tmax = jnp.asarray(_TMAX)
    p2 = pc2[0]
    p1 = pc1[0]
    fl = pred_flow[0]
    kci = jnp.clip(jnp.floor((p2 - minr) / _CELL).astype(jnp.int32), 0, tmax)
    deformed = p1 + fl
    qci = jnp.clip(jnp.floor((deformed - minr) / _CELL).astype(jnp.int32),
                   0, tmax)
    kc = kci.astype(jnp.float32)
    qc = qci.astype(jnp.float32)

    qr = qc.astype(jnp.bfloat16).astype(jnp.float32)
    kr = kc.astype(jnp.bfloat16).astype(jnp.float32)
    kni = jnp.sum(kci * kci, axis=1)
    knH = ((kni >> 11) << 11).astype(jnp.float32)
    knM = (((kni >> 3) & 0xFF) << 3).astype(jnp.float32)
    knL = (kni & 0x7).astype(jnp.float32)
    qaug_a = jnp.zeros((_NQ, 8), jnp.float32)
    qaug_a = qaug_a.at[:_N, 0:3].set(-2.0 * qr)
    qaug_a = qaug_a.at[:_N, 3:6].set(1.0)
    kaug_a = _pad_cols([kr[:, 0], kr[:, 1], kr[:, 2], knH, knM, knL])
    kaug_a = kaug_a.at[3, _N:].set(_PAD_KN)

    kh = (kci >> 4).astype(jnp.float32)
    kl = (kci & 0xF).astype(jnp.float32)
    kn6i = jnp.sum((kci >> 4) ** 2 + (kci & 0xF) ** 2, axis=1)
    kn6H = ((kn6i >> 4) << 4).astype(jnp.float32)
    kn6L = (kn6i & 0xF).astype(jnp.float32)
    qaug_b = jnp.zeros((_NQ, 8), jnp.float32)
    qaug_b = qaug_b.at[:_N, 0].set(-2.0 * kh[:, 0])
    qaug_b = qaug_b.at[:_N, 1].set(-2.0 * kl[:, 0])
    qaug_b = qaug_b.at[:_N, 2].set(-2.0 * kh[:, 1])
    qaug_b = qaug_b.at[:_N, 3].set(-2.0 * kl[:, 1])
    qaug_b = qaug_b.at[:_N, 4].set(-2.0 * kh[:, 2])
    qaug_b = qaug_b.at[:_N, 5].set(-2.0 * kl[:, 2])
    qaug_b = qaug_b.at[:_N, 6:8].set(1.0)
    kaug_b = _pad_cols([kh[:, 0], kl[:, 0], kh[:, 1], kl[:, 1],
                        kh[:, 2], kl[:, 2], kn6H, kn6L])
    kaug_b = kaug_b.at[6, _N:].set(_PAD_KN)
    nqn6 = jnp.zeros((_NQ, 1), jnp.float32)
    nqn6 = nqn6.at[:_N, 0].set(-kn6i.astype(jnp.float32))

    jstar = _nn_a(qaug_a.astype(jnp.bfloat16),
                  kaug_a.astype(jnp.bfloat16)).reshape(_NQ)
    rep = _nn_b(qaug_b.astype(jnp.bfloat16),
                kaug_b.astype(jnp.bfloat16), nqn6).reshape(_NQ)

    d2, nn = _make_sc_gather()(
        jstar, rep,
        p1[:, 0], p1[:, 1], p1[:, 2],
        fl[:, 0], fl[:, 1], fl[:, 2],
        p2[:, 0], p2[:, 1], p2[:, 2])

    dist = jnp.sqrt(d2[:_N]).reshape(1, _N)
    return dist, nn[:_N]
